# baseline (speedup 1.0000x reference)
import functools

import jax
import jax.numpy as jnp
from jax import lax
from jax.experimental import pallas as pl
from jax.experimental.pallas import tpu as pltpu
from jax.experimental.pallas import tpu_sc as plsc

NU = 5000
NI = 5000
NN = NU + NI
EDGES = 160000
DIM = 256
Q = 64
HOPS = 3

C = 128
E_PAD = 163840
NCHT = E_PAD // C
NCH = NCHT // 16
NP = 10240
RPT = NP // 16


def _body(x0, idxs, rowsc, valsc, zeros, o1, o2, o3,
          acc, idx2d_a, idx2d_b, rows2d, vals2d,
          gath0, gath1, sbuf0, sbuf1, gsem0, gsem1, ssem0, ssem1):
    c = lax.axis_index("c")
    s = lax.axis_index("s")
    cb = s * NCH
    rb = s * RPT

    pltpu.sync_copy(idxs.at[pl.ds((2 * c) * NCHT + cb, NCH)], idx2d_a)
    pltpu.sync_copy(idxs.at[pl.ds((2 * c + 1) * NCHT + cb, NCH)], idx2d_b)
    pltpu.sync_copy(rowsc.at[pl.ds(cb, NCH)], rows2d)
    pltpu.sync_copy(valsc.at[pl.ds(cb, NCH)], vals2d)

    srcs = [x0, o1, o2]
    dsts = [o1, o2, o3]
    for h in range(HOPS):
        for q, idx2d in enumerate((idx2d_a, idx2d_b)):
            src = srcs[h]
            qb = (2 * c + q) * NP

            pltpu.sync_copy(zeros.at[pl.ds(rb, RPT)], acc.at[pl.ds(rb, RPT)])
            plsc.subcore_barrier()

            def scale(k, gath, sbuf):
                for g in range(C // 16):
                    val16 = vals2d[k, pl.ds(g * 16, 16)]
                    for j in range(16):
                        bv = jnp.full((16,), val16[j], jnp.float32)
                        e = g * 16 + j
                        for f in range(Q // 16):
                            sl = (e, pl.ds(f * 16, 16))
                            sbuf[sl] = gath[sl] * bv

            pltpu.async_copy(src.at[idx2d.at[0]], gath0, gsem0)
            pltpu.async_copy(src.at[idx2d.at[1]], gath1, gsem1)

            bufs = ((gath0, sbuf0, gsem0, ssem0),
                    (gath1, sbuf1, gsem1, ssem1))

            def pair(p, _):
                for b, (gath, sbuf, gsem, ssem) in enumerate(bufs):
                    k = 2 * p + b
                    pltpu.make_async_copy(
                        src.at[pl.ds(0, C)], gath, gsem).wait()

                    @pl.when(k >= 2)
                    def _():
                        pltpu.make_async_copy(
                            zeros.at[pl.ds(0, C)], sbuf, ssem).wait()

                    scale(k, gath, sbuf)
                    pltpu.async_copy(
                        sbuf, acc.at[rows2d.at[k]], ssem, add=True)

                    @pl.when(k < NCH - 2)
                    def _():
                        pltpu.async_copy(
                            src.at[idx2d.at[k + 2]], gath, gsem)
                return 0

            lax.fori_loop(0, NCH // 2, pair, 0)
            pltpu.make_async_copy(zeros.at[pl.ds(0, C)], sbuf0, ssem0).wait()
            pltpu.make_async_copy(zeros.at[pl.ds(0, C)], sbuf1, ssem1).wait()
            plsc.subcore_barrier()
            pltpu.sync_copy(acc.at[pl.ds(rb, RPT)],
                            dsts[h].at[pl.ds(qb + rb, RPT)])


_lgn_sc = functools.partial(
    pl.kernel,
    out_type=[jax.ShapeDtypeStruct((4 * NP, Q), jnp.float32)] * HOPS,
    mesh=plsc.VectorSubcoreMesh(core_axis_name="c", subcore_axis_name="s"),
    compiler_params=pltpu.CompilerParams(use_tc_tiling_on_sc=False),
    scratch_types=[
        pltpu.VMEM_SHARED((NP, Q), jnp.float32),
        pltpu.VMEM((NCH, C), jnp.int32),
        pltpu.VMEM((NCH, C), jnp.int32),
        pltpu.VMEM((NCH, C), jnp.int32),
        pltpu.VMEM((NCH, C), jnp.float32),
        pltpu.VMEM((C, Q), jnp.float32),
        pltpu.VMEM((C, Q), jnp.float32),
        pltpu.VMEM((C, Q), jnp.float32),
        pltpu.VMEM((C, Q), jnp.float32),
        pltpu.SemaphoreType.DMA,
        pltpu.SemaphoreType.DMA,
        pltpu.SemaphoreType.DMA,
        pltpu.SemaphoreType.DMA,
    ],
)(_body)


def kernel(user_embed, item_embed, rows, cols, vals):
    all_embed = jnp.concatenate([user_embed, item_embed], axis=0)
    rpad = jnp.zeros((NP - NN, Q), jnp.float32)
    x0 = jnp.concatenate(
        [part for i in range(4)
         for part in (all_embed[:, i * Q:(i + 1) * Q], rpad)], axis=0)

    pad = E_PAD - EDGES
    rows_p = jnp.concatenate([rows.astype(jnp.int32),
                              jnp.zeros((pad,), jnp.int32)])
    cols_p = jnp.concatenate([cols.astype(jnp.int32),
                              jnp.zeros((pad,), jnp.int32)])
    vals_p = jnp.concatenate([vals, jnp.zeros((pad,), jnp.float32)])

    idxs = jnp.concatenate(
        [cols_p + i * NP for i in range(4)]).reshape(4 * NCHT, C)
    rowsc = rows_p.reshape(NCHT, C)
    valsc = vals_p.reshape(NCHT, C)
    zeros = jnp.zeros((NP, Q), jnp.float32)

    o1, o2, o3 = _lgn_sc(x0, idxs, rowsc, valsc, zeros)

    def unsplit(b):
        return jnp.concatenate(
            [b[i * NP:i * NP + NN] for i in range(4)], axis=1)

    embs = jnp.stack(
        [all_embed, unsplit(o1), unsplit(o2), unsplit(o3)], axis=1)
    return embs[:NU, :], embs[NU:, :]


# speedup vs baseline: 1.5478x; 1.5478x over previous
import functools

import jax
import jax.numpy as jnp
from jax import lax
from jax.experimental import pallas as pl
from jax.experimental.pallas import tpu as pltpu
from jax.experimental.pallas import tpu_sc as plsc

NU = 5000
NI = 5000
NN = NU + NI
EDGES = 160000
DIM = 256
Q = 64
HOPS = 3

C = 128
E_PAD = 163840
NCHT = E_PAD // C
NCH = NCHT // 16
B = 20
NB = NCH // B
NP = 10240
RPT = NP // 16


def _body(x0, idxs, rowsc, valsc, zeros, out,
          table, acc,
          idxB0, rowsB0, valsB0, idxB1, rowsB1, valsB1,
          gath0, gath1, sbuf0, sbuf1,
          gsem0, gsem1, ssem0, ssem1, bsem0, bsem1):
    c = lax.axis_index("c")
    s = lax.axis_index("s")
    cb = s * NCH
    rb = s * RPT

    eblk = ((idxB0, rowsB0, valsB0, bsem0), (idxB1, rowsB1, valsB1, bsem1))
    gbufs = ((gath0, sbuf0, gsem0, ssem0), (gath1, sbuf1, gsem1, ssem1))

    def load_block(bi, half):
        iB, rB, vB, bsem = eblk[half]
        pltpu.async_copy(idxs.at[pl.ds(cb + bi * B, B)], iB, bsem)
        pltpu.async_copy(rowsc.at[pl.ds(cb + bi * B, B)], rB, bsem)
        pltpu.async_copy(valsc.at[pl.ds(cb + bi * B, B)], vB, bsem)

    def wait_block(half):
        iB, rB, vB, bsem = eblk[half]
        pltpu.make_async_copy(idxs.at[pl.ds(0, B)], iB, bsem).wait()
        pltpu.make_async_copy(rowsc.at[pl.ds(0, B)], rB, bsem).wait()
        pltpu.make_async_copy(valsc.at[pl.ds(0, B)], vB, bsem).wait()

    def scale(vB, kk, gath, sbuf):
        def grp(g, _):
            val16 = vB[kk, pl.ds(g * 16, 16)]
            for j in range(16):
                bv = jnp.full((16,), val16[j], jnp.float32)
                e = g * 16 + j
                for f in range(Q // 16):
                    sl = (e, pl.ds(f * 16, 16))
                    sbuf[sl] = gath[sl] * bv
            return 0

        lax.fori_loop(0, C // 16, grp, 0)

    def chunk(half, kk, wait_sc, par):
        iB, rB, vB, _ = eblk[half]
        gath, sbuf, gsem, ssem = gbufs[par]
        pltpu.make_async_copy(table.at[pl.ds(0, C)], gath, gsem).wait()

        @pl.when(wait_sc)
        def _():
            pltpu.make_async_copy(zeros.at[pl.ds(0, C)], sbuf, ssem).wait()

        scale(vB, kk, gath, sbuf)
        pltpu.async_copy(sbuf, acc.at[rB.at[kk]], ssem, add=True)

        @pl.when(kk < B - 2)
        def _():
            pltpu.async_copy(table.at[iB.at[kk + 2]], gath, gsem)

    def do_pass(h, q):
        qoff = (2 * c + q) * NP

        @pl.when(h == 0)
        def _():
            pltpu.sync_copy(x0.at[pl.ds(qoff + rb, RPT)],
                            table.at[pl.ds(rb, RPT)])

        @pl.when(h > 0)
        def _():
            pltpu.sync_copy(out.at[h - 1, pl.ds(qoff + rb, RPT)],
                            table.at[pl.ds(rb, RPT)])

        pltpu.sync_copy(zeros.at[pl.ds(rb, RPT)], acc.at[pl.ds(rb, RPT)])
        load_block(0, 0)
        plsc.subcore_barrier()

        def block(bi, half):
            iB, _, _, _ = eblk[half]
            wait_block(half)
            pltpu.async_copy(table.at[iB.at[0]], gath0, gsem0)
            pltpu.async_copy(table.at[iB.at[1]], gath1, gsem1)
            chunk(half, 0, bi >= 1, 0)
            chunk(half, 1, bi >= 1, 1)

            @pl.when(bi < NB - 1)
            def _():
                load_block(bi + 1, 1 - half)

            def pairs(p, _):
                chunk(half, 2 * p, True, 0)
                chunk(half, 2 * p + 1, True, 1)
                return 0

            lax.fori_loop(1, B // 2, pairs, 0)

        def superblock(sb, _):
            block(2 * sb, 0)
            block(2 * sb + 1, 1)
            return 0

        lax.fori_loop(0, NB // 2, superblock, 0)
        pltpu.make_async_copy(zeros.at[pl.ds(0, C)], sbuf0, ssem0).wait()
        pltpu.make_async_copy(zeros.at[pl.ds(0, C)], sbuf1, ssem1).wait()
        plsc.subcore_barrier()
        pltpu.sync_copy(acc.at[pl.ds(rb, RPT)],
                        out.at[h, pl.ds(qoff + rb, RPT)])

    def hop(h, _):
        lax.fori_loop(0, 2, lambda q, _: (do_pass(h, q), 0)[1], 0)
        return 0

    lax.fori_loop(0, HOPS, hop, 0)


_lgn_sc = functools.partial(
    pl.kernel,
    out_type=jax.ShapeDtypeStruct((HOPS, 4 * NP, Q), jnp.float32),
    mesh=plsc.VectorSubcoreMesh(core_axis_name="c", subcore_axis_name="s"),
    compiler_params=pltpu.CompilerParams(use_tc_tiling_on_sc=False),
    scratch_types=[
        pltpu.VMEM_SHARED((NP, Q), jnp.float32),
        pltpu.VMEM_SHARED((NP, Q), jnp.float32),
        pltpu.VMEM((B, C), jnp.int32),
        pltpu.VMEM((B, C), jnp.int32),
        pltpu.VMEM((B, C), jnp.float32),
        pltpu.VMEM((B, C), jnp.int32),
        pltpu.VMEM((B, C), jnp.int32),
        pltpu.VMEM((B, C), jnp.float32),
        pltpu.VMEM((C, Q), jnp.float32),
        pltpu.VMEM((C, Q), jnp.float32),
        pltpu.VMEM((C, Q), jnp.float32),
        pltpu.VMEM((C, Q), jnp.float32),
        pltpu.SemaphoreType.DMA,
        pltpu.SemaphoreType.DMA,
        pltpu.SemaphoreType.DMA,
        pltpu.SemaphoreType.DMA,
        pltpu.SemaphoreType.DMA,
        pltpu.SemaphoreType.DMA,
    ],
)(_body)


def kernel(user_embed, item_embed, rows, cols, vals):
    all_embed = jnp.concatenate([user_embed, item_embed], axis=0)
    rpad = jnp.zeros((NP - NN, Q), jnp.float32)
    x0 = jnp.concatenate(
        [part for i in range(4)
         for part in (all_embed[:, i * Q:(i + 1) * Q], rpad)], axis=0)

    pad = E_PAD - EDGES
    rows_p = jnp.concatenate([rows.astype(jnp.int32),
                              jnp.zeros((pad,), jnp.int32)])
    cols_p = jnp.concatenate([cols.astype(jnp.int32),
                              jnp.zeros((pad,), jnp.int32)])
    vals_p = jnp.concatenate([vals, jnp.zeros((pad,), jnp.float32)])

    idxs = cols_p.reshape(NCHT, C)
    rowsc = rows_p.reshape(NCHT, C)
    valsc = vals_p.reshape(NCHT, C)
    zeros = jnp.zeros((NP, Q), jnp.float32)

    out = _lgn_sc(x0, idxs, rowsc, valsc, zeros)

    def unsplit(b):
        return jnp.concatenate(
            [b[i * NP:i * NP + NN] for i in range(4)], axis=1)

    embs = jnp.stack(
        [all_embed, unsplit(out[0]), unsplit(out[1]), unsplit(out[2])],
        axis=1)
    return embs[:NU, :], embs[NU:, :]


# speedup vs baseline: 1.5490x; 1.0008x over previous
import functools

import jax
import jax.numpy as jnp
from jax import lax
from jax.experimental import pallas as pl
from jax.experimental.pallas import tpu as pltpu
from jax.experimental.pallas import tpu_sc as plsc

NU = 5000
NI = 5000
NN = NU + NI
EDGES = 160000
DIM = 256
Q = 64
HOPS = 3

C = 128
E_PAD = 163840
NCHT = E_PAD // C
NCH = NCHT // 16
B = 20
NB = NCH // B
NP = 10240
RPT = NP // 16


def _body(x0, idxs, rowsc, valsc, zeros, out,
          table, acc,
          idxB0, rowsB0, valsB0, idxB1, rowsB1, valsB1,
          gath0, gath1, sbuf0, sbuf1,
          gsem0, gsem1, ssem0, ssem1, bsem0, bsem1):
    c = lax.axis_index("c")
    s = lax.axis_index("s")
    cb = s * NCH
    rb = s * RPT

    eblk = ((idxB0, rowsB0, valsB0, bsem0), (idxB1, rowsB1, valsB1, bsem1))
    gbufs = ((gath0, sbuf0, gsem0, ssem0), (gath1, sbuf1, gsem1, ssem1))

    def load_block(bi, half):
        iB, rB, vB, bsem = eblk[half]
        pltpu.async_copy(idxs.at[pl.ds(cb + bi * B, B)], iB, bsem)
        pltpu.async_copy(rowsc.at[pl.ds(cb + bi * B, B)], rB, bsem)
        pltpu.async_copy(valsc.at[pl.ds(cb + bi * B, B)], vB, bsem)

    def wait_block(half):
        iB, rB, vB, bsem = eblk[half]
        pltpu.make_async_copy(idxs.at[pl.ds(0, B)], iB, bsem).wait()
        pltpu.make_async_copy(rowsc.at[pl.ds(0, B)], rB, bsem).wait()
        pltpu.make_async_copy(valsc.at[pl.ds(0, B)], vB, bsem).wait()

    def scale(vB, kk, gath, sbuf):
        def grp(g, _):
            val16 = vB[kk, pl.ds(g * 16, 16)]
            for j in range(16):
                bv = jnp.full((16,), val16[j], jnp.float32)
                e = g * 16 + j
                for f in range(Q // 16):
                    sl = (e, pl.ds(f * 16, 16))
                    sbuf[sl] = gath[sl] * bv
            return 0

        lax.fori_loop(0, C // 16, grp, 0)

    def chunk(half, kk, wait_sc, par):
        iB, rB, vB, _ = eblk[half]
        gath, sbuf, gsem, ssem = gbufs[par]
        pltpu.make_async_copy(table.at[pl.ds(0, C)], gath, gsem).wait()

        @pl.when(wait_sc)
        def _():
            pltpu.make_async_copy(zeros.at[pl.ds(0, C)], sbuf, ssem).wait()

        scale(vB, kk, gath, sbuf)
        pltpu.async_copy(sbuf.at[pl.ds(0, C // 2)],
                         acc.at[rB.at[kk, pl.ds(0, C // 2)]], ssem, add=True)
        pltpu.async_copy(sbuf.at[pl.ds(C // 2, C // 2)],
                         acc.at[rB.at[kk, pl.ds(C // 2, C // 2)]],
                         ssem, add=True)

        @pl.when(kk < B - 2)
        def _():
            pltpu.async_copy(table.at[iB.at[kk + 2, pl.ds(0, C // 2)]],
                             gath.at[pl.ds(0, C // 2)], gsem)
            pltpu.async_copy(table.at[iB.at[kk + 2, pl.ds(C // 2, C // 2)]],
                             gath.at[pl.ds(C // 2, C // 2)], gsem)

    def do_pass(h, q):
        qoff = (2 * c + q) * NP

        @pl.when(h == 0)
        def _():
            pltpu.sync_copy(x0.at[pl.ds(qoff + rb, RPT)],
                            table.at[pl.ds(rb, RPT)])

        @pl.when(h > 0)
        def _():
            pltpu.sync_copy(out.at[h - 1, pl.ds(qoff + rb, RPT)],
                            table.at[pl.ds(rb, RPT)])

        pltpu.sync_copy(zeros.at[pl.ds(rb, RPT)], acc.at[pl.ds(rb, RPT)])
        load_block(0, 0)
        plsc.subcore_barrier()

        def block(bi, half):
            iB, _, _, _ = eblk[half]
            wait_block(half)
            pltpu.async_copy(table.at[iB.at[0, pl.ds(0, C // 2)]],
                             gath0.at[pl.ds(0, C // 2)], gsem0)
            pltpu.async_copy(table.at[iB.at[0, pl.ds(C // 2, C // 2)]],
                             gath0.at[pl.ds(C // 2, C // 2)], gsem0)
            pltpu.async_copy(table.at[iB.at[1, pl.ds(0, C // 2)]],
                             gath1.at[pl.ds(0, C // 2)], gsem1)
            pltpu.async_copy(table.at[iB.at[1, pl.ds(C // 2, C // 2)]],
                             gath1.at[pl.ds(C // 2, C // 2)], gsem1)
            chunk(half, 0, bi >= 1, 0)
            chunk(half, 1, bi >= 1, 1)

            @pl.when(bi < NB - 1)
            def _():
                load_block(bi + 1, 1 - half)

            def pairs(p, _):
                chunk(half, 2 * p, True, 0)
                chunk(half, 2 * p + 1, True, 1)
                return 0

            lax.fori_loop(1, B // 2, pairs, 0)

        def superblock(sb, _):
            block(2 * sb, 0)
            block(2 * sb + 1, 1)
            return 0

        lax.fori_loop(0, NB // 2, superblock, 0)
        pltpu.make_async_copy(zeros.at[pl.ds(0, C)], sbuf0, ssem0).wait()
        pltpu.make_async_copy(zeros.at[pl.ds(0, C)], sbuf1, ssem1).wait()
        plsc.subcore_barrier()
        pltpu.sync_copy(acc.at[pl.ds(rb, RPT)],
                        out.at[h, pl.ds(qoff + rb, RPT)])

    def hop(h, _):
        lax.fori_loop(0, 2, lambda q, _: (do_pass(h, q), 0)[1], 0)
        return 0

    lax.fori_loop(0, HOPS, hop, 0)


_lgn_sc = functools.partial(
    pl.kernel,
    out_type=jax.ShapeDtypeStruct((HOPS, 4 * NP, Q), jnp.float32),
    mesh=plsc.VectorSubcoreMesh(core_axis_name="c", subcore_axis_name="s"),
    compiler_params=pltpu.CompilerParams(use_tc_tiling_on_sc=False),
    scratch_types=[
        pltpu.VMEM_SHARED((NP, Q), jnp.float32),
        pltpu.VMEM_SHARED((NP, Q), jnp.float32),
        pltpu.VMEM((B, C), jnp.int32),
        pltpu.VMEM((B, C), jnp.int32),
        pltpu.VMEM((B, C), jnp.float32),
        pltpu.VMEM((B, C), jnp.int32),
        pltpu.VMEM((B, C), jnp.int32),
        pltpu.VMEM((B, C), jnp.float32),
        pltpu.VMEM((C, Q), jnp.float32),
        pltpu.VMEM((C, Q), jnp.float32),
        pltpu.VMEM((C, Q), jnp.float32),
        pltpu.VMEM((C, Q), jnp.float32),
        pltpu.SemaphoreType.DMA,
        pltpu.SemaphoreType.DMA,
        pltpu.SemaphoreType.DMA,
        pltpu.SemaphoreType.DMA,
        pltpu.SemaphoreType.DMA,
        pltpu.SemaphoreType.DMA,
    ],
)(_body)


def kernel(user_embed, item_embed, rows, cols, vals):
    all_embed = jnp.concatenate([user_embed, item_embed], axis=0)
    rpad = jnp.zeros((NP - NN, Q), jnp.float32)
    x0 = jnp.concatenate(
        [part for i in range(4)
         for part in (all_embed[:, i * Q:(i + 1) * Q], rpad)], axis=0)

    pad = E_PAD - EDGES
    rows_p = jnp.concatenate([rows.astype(jnp.int32),
                              jnp.zeros((pad,), jnp.int32)])
    cols_p = jnp.concatenate([cols.astype(jnp.int32),
                              jnp.zeros((pad,), jnp.int32)])
    vals_p = jnp.concatenate([vals, jnp.zeros((pad,), jnp.float32)])

    idxs = cols_p.reshape(NCHT, C)
    rowsc = rows_p.reshape(NCHT, C)
    valsc = vals_p.reshape(NCHT, C)
    zeros = jnp.zeros((NP, Q), jnp.float32)

    out = _lgn_sc(x0, idxs, rowsc, valsc, zeros)

    def unsplit(b):
        return jnp.concatenate(
            [b[i * NP:i * NP + NN] for i in range(4)], axis=1)

    embs = jnp.stack(
        [all_embed, unsplit(out[0]), unsplit(out[1]), unsplit(out[2])],
        axis=1)
    return embs[:NU, :], embs[NU:, :]
